# baseline (speedup 1.0000x reference)
import dataclasses
import functools

import jax
import jax.numpy as jnp
from jax import lax
from jax.experimental import pallas as pl
from jax.experimental.pallas import tpu as pltpu
from jax.experimental.pallas import tpu_sc as plsc

VOCAB = 50265
HIDDEN = 1024
PAD = 1
EPS = 1e-12

NC = 2
NS = 16
LANES = 16
NW = NC * NS
B, S = 4, 2048
TOKENS = B * S
TPW = TOKENS // NW
SEGS_PER_ROW = S // TPW
G = 16
NBUF = 3


def _sc_gather(input_ids, word_emb, pos_emb, tok_off, ntok):
    tpw = ntok // NW
    nchunk = tpw // G
    mesh = plsc.VectorSubcoreMesh(core_axis_name="c", subcore_axis_name="s",
                                  num_cores=NC, num_subcores=NS)
    cp = pltpu.CompilerParams()
    if "needs_layout_passes" in pltpu.CompilerParams.__dataclass_fields__:
        cp = dataclasses.replace(cp, needs_layout_passes=False)

    row_bufs = [pltpu.VMEM((G, HIDDEN), jnp.float32)] * (2 * NBUF)
    sems = [pltpu.SemaphoreType.DMA] * (4 * NBUF)

    @pl.kernel(
        compiler_params=cp,
        out_type=(jax.ShapeDtypeStruct((ntok, HIDDEN), jnp.float32),
                  jax.ShapeDtypeStruct((ntok, HIDDEN), jnp.float32)),
        mesh=mesh,
        scratch_types=[
            pltpu.VMEM((S,), jnp.int32),
            pltpu.VMEM((tpw,), jnp.int32),
        ] + row_bufs + sems,
    )
    def k(ids_hbm, word_hbm, pos_hbm, wout_hbm, pout_hbm, ids_v, pidx_v,
          *bufs_and_sems):
        wid = lax.axis_index("s") * NC + lax.axis_index("c")
        tok0 = tok_off + wid * tpw
        row = tok0 // S
        seg_off = tok0 % S
        base = wid * tpw

        wbufs = list(bufs_and_sems[:NBUF])
        pbufs = list(bufs_and_sems[NBUF:2 * NBUF])
        wsems = list(bufs_and_sems[2 * NBUF:3 * NBUF])
        psems = list(bufs_and_sems[3 * NBUF:4 * NBUF])
        wosems = list(bufs_and_sems[4 * NBUF:5 * NBUF])
        posems = list(bufs_and_sems[5 * NBUF:6 * NBUF])

        pltpu.sync_copy(ids_hbm.at[row], ids_v)

        word_early = []
        for s in range(min(NBUF, nchunk)):
            widx = ids_v.at[pl.ds(seg_off + s * G, G)]
            word_early.append(
                pltpu.async_copy(word_hbm.at[widx], wbufs[s], wsems[s]))

        one = jnp.int32(1)
        zero = jnp.int32(0)

        def pre_body(i, acc):
            v = ids_v[pl.ds(i * LANES, LANES)]
            return acc + jnp.where(v != PAD, one, zero)

        acc = lax.fori_loop(0, seg_off // LANES, pre_body,
                            jnp.zeros((LANES,), jnp.int32))
        prefix = jnp.sum(acc)

        def pos_body(k_, carry):
            v = ids_v[pl.ds(seg_off + k_ * LANES, LANES)]
            m = jnp.where(v != PAD, one, zero)
            c = plsc.cumsum(m)
            pidx_v[pl.ds(k_ * LANES, LANES)] = (carry + c) * m + PAD
            return carry + jnp.sum(m)

        lax.fori_loop(0, tpw // LANES, pos_body, prefix)

        def issue_gathers(g, s):
            widx = ids_v.at[pl.ds(seg_off + g * G, G)]
            pidx = pidx_v.at[pl.ds(g * G, G)]
            return (pltpu.async_copy(word_hbm.at[widx], wbufs[s], wsems[s]),
                    pltpu.async_copy(pos_hbm.at[pidx], pbufs[s], psems[s]))

        pending = [
            (word_early[s],
             pltpu.async_copy(pos_hbm.at[pidx_v.at[pl.ds(s * G, G)]],
                              pbufs[s], psems[s]))
            for s in range(min(NBUF, nchunk))]
        pending += [None] * (NBUF - len(pending))
        pending_out = [None] * NBUF
        for g in range(nchunk):
            s = g % NBUF
            wc, pc = pending[s]
            wc.wait()
            pc.wait()
            dst = pl.ds(base + g * G, G)
            pending_out[s] = (
                pltpu.async_copy(wbufs[s], wout_hbm.at[dst], wosems[s]),
                pltpu.async_copy(pbufs[s], pout_hbm.at[dst], posems[s]))
            if g + NBUF < nchunk:
                oc, oc2 = pending_out[s]
                oc.wait()
                oc2.wait()
                pending[s] = issue_gathers(g + NBUF, s)
                pending_out[s] = None

        for s in range(NBUF):
            if pending_out[s] is not None:
                oc, oc2 = pending_out[s]
                oc.wait()
                oc2.wait()

    return k(input_ids, word_emb, pos_emb)


def _ln_body(w_ref, p_ref, t_ref, g_ref, b_ref, o_ref):
    x = w_ref[...] + p_ref[...] + t_ref[...]
    mu = jnp.mean(x, axis=-1, keepdims=True)
    d = x - mu
    var = jnp.mean(d * d, axis=-1, keepdims=True)
    o_ref[...] = d * lax.rsqrt(var + EPS) * g_ref[...] + b_ref[...]


def _tc_layernorm(wrows, prows, type_row, ln_w, ln_b):
    ntok = wrows.shape[0]
    blk = 1024
    return pl.pallas_call(
        _ln_body,
        grid=(ntok // blk,),
        in_specs=[
            pl.BlockSpec((blk, HIDDEN), lambda i: (i, 0)),
            pl.BlockSpec((blk, HIDDEN), lambda i: (i, 0)),
            pl.BlockSpec((1, HIDDEN), lambda i: (0, 0)),
            pl.BlockSpec((1, HIDDEN), lambda i: (0, 0)),
            pl.BlockSpec((1, HIDDEN), lambda i: (0, 0)),
        ],
        out_specs=pl.BlockSpec((blk, HIDDEN), lambda i: (i, 0)),
        out_shape=jax.ShapeDtypeStruct((ntok, HIDDEN), jnp.float32),
    )(wrows, prows, type_row, ln_w, ln_b)


def kernel(input_ids, word_emb, pos_emb, type_emb, ln_w, ln_b):
    ids = input_ids.astype(jnp.int32)
    trow = type_emb[0:1]
    lw = ln_w.reshape(1, HIDDEN)
    lb = ln_b.reshape(1, HIDDEN)
    wrows, prows = _sc_gather(ids, word_emb, pos_emb, 0, TOKENS)
    out = _tc_layernorm(wrows, prows, trow, lw, lb)
    return out.reshape(B, S, HIDDEN)


# speedup vs baseline: 1.0092x; 1.0092x over previous
import dataclasses
import functools

import jax
import jax.numpy as jnp
from jax import lax
from jax.experimental import pallas as pl
from jax.experimental.pallas import tpu as pltpu
from jax.experimental.pallas import tpu_sc as plsc

VOCAB = 50265
HIDDEN = 1024
PAD = 1
EPS = 1e-12

NC = 2
NS = 16
LANES = 16
NW = NC * NS
B, S = 4, 2048
TOKENS = B * S
TPW = TOKENS // NW
SEGS_PER_ROW = S // TPW
G = 16
NBUF = 3


def _sc_gather(input_ids, word_emb, pos_emb, tok_off, ntok):
    tpw = ntok // NW
    nchunk = tpw // G
    mesh = plsc.VectorSubcoreMesh(core_axis_name="c", subcore_axis_name="s",
                                  num_cores=NC, num_subcores=NS)
    cp = pltpu.CompilerParams()
    if "needs_layout_passes" in pltpu.CompilerParams.__dataclass_fields__:
        cp = dataclasses.replace(cp, needs_layout_passes=False)

    row_bufs = [pltpu.VMEM((G, HIDDEN), jnp.float32)] * (2 * NBUF)
    sems = [pltpu.SemaphoreType.DMA] * (4 * NBUF)

    @pl.kernel(
        compiler_params=cp,
        out_type=(jax.ShapeDtypeStruct((ntok, HIDDEN), jnp.float32),
                  jax.ShapeDtypeStruct((ntok, HIDDEN), jnp.float32)),
        mesh=mesh,
        scratch_types=[
            pltpu.VMEM((S,), jnp.int32),
            pltpu.VMEM((tpw,), jnp.int32),
        ] + row_bufs + sems,
    )
    def k(ids_hbm, word_hbm, pos_hbm, wout_hbm, pout_hbm, ids_v, pidx_v,
          *bufs_and_sems):
        wid = lax.axis_index("s") * NC + lax.axis_index("c")
        tok0 = tok_off + wid * tpw
        row = tok0 // S
        seg_off = tok0 % S
        base = wid * tpw

        wbufs = list(bufs_and_sems[:NBUF])
        pbufs = list(bufs_and_sems[NBUF:2 * NBUF])
        wsems = list(bufs_and_sems[2 * NBUF:3 * NBUF])
        psems = list(bufs_and_sems[3 * NBUF:4 * NBUF])
        wosems = list(bufs_and_sems[4 * NBUF:5 * NBUF])
        posems = list(bufs_and_sems[5 * NBUF:6 * NBUF])

        pltpu.sync_copy(ids_hbm.at[row], ids_v)

        one = jnp.int32(1)
        zero = jnp.int32(0)

        def pre_body(i, acc):
            v = ids_v[pl.ds(i * LANES, LANES)]
            return acc + jnp.where(v != PAD, one, zero)

        acc = lax.fori_loop(0, seg_off // LANES, pre_body,
                            jnp.zeros((LANES,), jnp.int32))
        prefix = jnp.sum(acc)

        def pos_body(k_, carry):
            v = ids_v[pl.ds(seg_off + k_ * LANES, LANES)]
            m = jnp.where(v != PAD, one, zero)
            c = plsc.cumsum(m)
            pidx_v[pl.ds(k_ * LANES, LANES)] = (carry + c) * m + PAD
            return carry + jnp.sum(m)

        lax.fori_loop(0, tpw // LANES, pos_body, prefix)

        def issue_gathers(g, s):
            widx = ids_v.at[pl.ds(seg_off + g * G, G)]
            pidx = pidx_v.at[pl.ds(g * G, G)]
            return (pltpu.async_copy(word_hbm.at[widx], wbufs[s], wsems[s]),
                    pltpu.async_copy(pos_hbm.at[pidx], pbufs[s], psems[s]))

        pending = [issue_gathers(s, s) for s in range(min(NBUF, nchunk))]
        pending += [None] * (NBUF - len(pending))
        pending_out = [None] * NBUF
        for g in range(nchunk):
            s = g % NBUF
            wc, pc = pending[s]
            wc.wait()
            pc.wait()
            dst = pl.ds(base + g * G, G)
            pending_out[s] = (
                pltpu.async_copy(wbufs[s], wout_hbm.at[dst], wosems[s]),
                pltpu.async_copy(pbufs[s], pout_hbm.at[dst], posems[s]))
            if g + NBUF < nchunk:
                oc, oc2 = pending_out[s]
                oc.wait()
                oc2.wait()
                pending[s] = issue_gathers(g + NBUF, s)
                pending_out[s] = None

        for s in range(NBUF):
            if pending_out[s] is not None:
                oc, oc2 = pending_out[s]
                oc.wait()
                oc2.wait()

    return k(input_ids, word_emb, pos_emb)


def _ln_body(w_ref, p_ref, t_ref, g_ref, b_ref, o_ref):
    x = w_ref[...] + p_ref[...] + t_ref[...]
    mu = jnp.mean(x, axis=-1, keepdims=True)
    d = x - mu
    var = jnp.mean(d * d, axis=-1, keepdims=True)
    o_ref[...] = d * lax.rsqrt(var + EPS) * g_ref[...] + b_ref[...]


def _tc_layernorm(wrows, prows, type_row, ln_w, ln_b):
    ntok = wrows.shape[0]
    blk = 2048
    return pl.pallas_call(
        _ln_body,
        grid=(ntok // blk,),
        in_specs=[
            pl.BlockSpec((blk, HIDDEN), lambda i: (i, 0)),
            pl.BlockSpec((blk, HIDDEN), lambda i: (i, 0)),
            pl.BlockSpec((1, HIDDEN), lambda i: (0, 0)),
            pl.BlockSpec((1, HIDDEN), lambda i: (0, 0)),
            pl.BlockSpec((1, HIDDEN), lambda i: (0, 0)),
        ],
        out_specs=pl.BlockSpec((blk, HIDDEN), lambda i: (i, 0)),
        out_shape=jax.ShapeDtypeStruct((ntok, HIDDEN), jnp.float32),
    )(wrows, prows, type_row, ln_w, ln_b)


def kernel(input_ids, word_emb, pos_emb, type_emb, ln_w, ln_b):
    ids = input_ids.astype(jnp.int32)
    trow = type_emb[0:1]
    lw = ln_w.reshape(1, HIDDEN)
    lb = ln_b.reshape(1, HIDDEN)
    wrows, prows = _sc_gather(ids, word_emb, pos_emb, 0, TOKENS)
    out = _tc_layernorm(wrows, prows, trow, lw, lb)
    return out.reshape(B, S, HIDDEN)


# speedup vs baseline: 1.0100x; 1.0008x over previous
import dataclasses

import jax
import jax.numpy as jnp
from jax import lax
from jax.experimental import pallas as pl
from jax.experimental.pallas import tpu as pltpu
from jax.experimental.pallas import tpu_sc as plsc

HIDDEN = 1024
PAD = 1
EPS = 1e-12

NC = 2
NS = 16
LANES = 16
NW = NC * NS
B, S = 4, 2048
TOKENS = B * S
G = 16
NBUF = 3


def _sc_gather(input_ids, word_emb, pos_emb, tok_off, ntok):
    tpw = ntok // NW
    nchunk = tpw // G
    mesh = plsc.VectorSubcoreMesh(core_axis_name="c", subcore_axis_name="s",
                                  num_cores=NC, num_subcores=NS)
    cp = pltpu.CompilerParams()
    if "needs_layout_passes" in pltpu.CompilerParams.__dataclass_fields__:
        cp = dataclasses.replace(cp, needs_layout_passes=False)

    row_bufs = [pltpu.VMEM((G, HIDDEN), jnp.float32)] * (2 * NBUF)
    sems = [pltpu.SemaphoreType.DMA] * (4 * NBUF)

    @pl.kernel(
        compiler_params=cp,
        out_type=(jax.ShapeDtypeStruct((ntok, HIDDEN), jnp.float32),
                  jax.ShapeDtypeStruct((ntok, HIDDEN), jnp.float32)),
        mesh=mesh,
        scratch_types=[
            pltpu.VMEM((S,), jnp.int32),
            pltpu.VMEM((tpw,), jnp.int32),
        ] + row_bufs + sems,
    )
    def k(ids_hbm, word_hbm, pos_hbm, wout_hbm, pout_hbm, ids_v, pidx_v,
          *bufs_and_sems):
        wid = lax.axis_index("s") * NC + lax.axis_index("c")
        tok0 = tok_off + wid * tpw
        row = tok0 // S
        seg_off = tok0 % S
        base = wid * tpw

        wbufs = list(bufs_and_sems[:NBUF])
        pbufs = list(bufs_and_sems[NBUF:2 * NBUF])
        wsems = list(bufs_and_sems[2 * NBUF:3 * NBUF])
        psems = list(bufs_and_sems[3 * NBUF:4 * NBUF])
        wosems = list(bufs_and_sems[4 * NBUF:5 * NBUF])
        posems = list(bufs_and_sems[5 * NBUF:6 * NBUF])

        pltpu.sync_copy(ids_hbm.at[row], ids_v)

        one = jnp.int32(1)
        zero = jnp.int32(0)

        def pre_body(i, acc):
            v = ids_v[pl.ds(i * LANES, LANES)]
            return acc + jnp.where(v != PAD, one, zero)

        acc = lax.fori_loop(0, seg_off // LANES, pre_body,
                            jnp.zeros((LANES,), jnp.int32))
        prefix = jnp.sum(acc)

        def pos_body(k_, carry):
            v = ids_v[pl.ds(seg_off + k_ * LANES, LANES)]
            m = jnp.where(v != PAD, one, zero)
            c = plsc.cumsum(m)
            pidx_v[pl.ds(k_ * LANES, LANES)] = (carry + c) * m + PAD
            return carry + jnp.sum(m)

        lax.fori_loop(0, tpw // LANES, pos_body, prefix)

        def issue_gathers(g, s):
            widx = ids_v.at[pl.ds(seg_off + g * G, G)]
            pidx = pidx_v.at[pl.ds(g * G, G)]
            return (pltpu.async_copy(word_hbm.at[widx], wbufs[s], wsems[s]),
                    pltpu.async_copy(pos_hbm.at[pidx], pbufs[s], psems[s]))

        pending = [issue_gathers(s, s) for s in range(min(NBUF, nchunk))]
        pending += [None] * (NBUF - len(pending))
        pending_out = [None] * NBUF
        for g in range(nchunk):
            s = g % NBUF
            wc, pc = pending[s]
            wc.wait()
            pc.wait()
            dst = pl.ds(base + g * G, G)
            pending_out[s] = (
                pltpu.async_copy(wbufs[s], wout_hbm.at[dst], wosems[s]),
                pltpu.async_copy(pbufs[s], pout_hbm.at[dst], posems[s]))
            if g + NBUF < nchunk:
                oc, oc2 = pending_out[s]
                oc.wait()
                oc2.wait()
                pending[s] = issue_gathers(g + NBUF, s)
                pending_out[s] = None

        for s in range(NBUF):
            if pending_out[s] is not None:
                oc, oc2 = pending_out[s]
                oc.wait()
                oc2.wait()

    return k(input_ids, word_emb, pos_emb)


def _ln_body(w_ref, p_ref, t_ref, g_ref, b_ref, o_ref):
    x = w_ref[...] + p_ref[...] + t_ref[...]
    mu = jnp.mean(x, axis=-1, keepdims=True)
    d = x - mu
    var = jnp.mean(d * d, axis=-1, keepdims=True)
    o_ref[...] = d * lax.rsqrt(var + EPS) * g_ref[...] + b_ref[...]


def _tc_layernorm(wrows, prows, type_row, ln_w, ln_b):
    ntok = wrows.shape[0]
    blk = 2048
    return pl.pallas_call(
        _ln_body,
        grid=(ntok // blk,),
        in_specs=[
            pl.BlockSpec((blk, HIDDEN), lambda i: (i, 0)),
            pl.BlockSpec((blk, HIDDEN), lambda i: (i, 0)),
            pl.BlockSpec((1, HIDDEN), lambda i: (0, 0)),
            pl.BlockSpec((1, HIDDEN), lambda i: (0, 0)),
            pl.BlockSpec((1, HIDDEN), lambda i: (0, 0)),
        ],
        out_specs=pl.BlockSpec((blk, HIDDEN), lambda i: (i, 0)),
        out_shape=jax.ShapeDtypeStruct((ntok, HIDDEN), jnp.float32),
    )(wrows, prows, type_row, ln_w, ln_b)


def kernel(input_ids, word_emb, pos_emb, type_emb, ln_w, ln_b):
    ids = input_ids.astype(jnp.int32)
    trow = type_emb[0:1]
    lw = ln_w.reshape(1, HIDDEN)
    lb = ln_b.reshape(1, HIDDEN)
    wrows, prows = _sc_gather(ids, word_emb, pos_emb, 0, TOKENS)
    out = _tc_layernorm(wrows, prows, trow, lw, lb)
    return out.reshape(B, S, HIDDEN)
